# baseline (speedup 1.0000x reference)
import functools

import jax
import jax.numpy as jnp
from jax import lax
from jax.experimental import pallas as pl
from jax.experimental.pallas import tpu as pltpu
from jax.experimental.pallas import tpu_sc as plsc

N = 10000
E = 320000
D = 128

NPAD = 10240
SENT = 10000
NTILES = 32
CHUNK = 128
EPT = 10240
NCHUNK = EPT // CHUNK
E_PAD = EPT * NTILES
ROWS_PER_TILE = NPAD // 16
R = 1024

_MESH = plsc.VectorSubcoreMesh(core_axis_name="c", subcore_axis_name="s")


def _zero_rows(rows):
    @pl.loop(0, CHUNK)
    def _r(r):
        @pl.loop(0, D // 16)
        def _c(j):
            rows[r, pl.ds(j * 16, 16)] = jnp.zeros((16,), jnp.float32)


@functools.partial(
    pl.kernel,
    out_type=jax.ShapeDtypeStruct((2 * NPAD,), jnp.float32),
    mesh=_MESH,
    scratch_types=[
        pltpu.VMEM((NCHUNK, 1, CHUNK), jnp.int32),
        pltpu.VMEM((CHUNK,), jnp.float32),
        pltpu.VMEM((ROWS_PER_TILE,), jnp.float32),
        pltpu.VMEM_SHARED((NPAD,), jnp.float32),
    ],
)
def _sc_degree(dst_hbm, out_hbm, dstb, ones, zb, deg):
    c = lax.axis_index("c")
    s = lax.axis_index("s")
    wid = c * 16 + s

    @pl.loop(0, CHUNK // 16)
    def _o(i):
        ones[pl.ds(i * 16, 16)] = jnp.full((16,), 1.0, jnp.float32)

    @pl.loop(0, ROWS_PER_TILE // 16)
    def _z(i):
        zb[pl.ds(i * 16, 16)] = jnp.zeros((16,), jnp.float32)

    pltpu.sync_copy(zb, deg.at[pl.ds(s * ROWS_PER_TILE, ROWS_PER_TILE)])
    pltpu.sync_copy(dst_hbm.at[pl.ds(wid * NCHUNK, NCHUNK)], dstb)
    plsc.subcore_barrier()

    @pl.loop(0, NCHUNK)
    def _k(k):
        pltpu.sync_copy(ones, deg.at[dstb.at[k, 0]], add=True)

    plsc.subcore_barrier()
    pltpu.sync_copy(
        deg.at[pl.ds(s * ROWS_PER_TILE, ROWS_PER_TILE)],
        out_hbm.at[pl.ds(c * NPAD + s * ROWS_PER_TILE, ROWS_PER_TILE)],
    )


@functools.partial(
    pl.kernel,
    out_type=jax.ShapeDtypeStruct((2 * NPAD, D), jnp.float32),
    mesh=_MESH,
    scratch_types=[
        pltpu.VMEM((3, 1, CHUNK), jnp.int32),
        pltpu.VMEM((NCHUNK, 1, CHUNK), jnp.int32),
        pltpu.VMEM((2, CHUNK, D), jnp.float32),
        pltpu.VMEM_SHARED((NPAD, D), jnp.float32),
        pltpu.SemaphoreType.DMA,
        pltpu.SemaphoreType.DMA,
        pltpu.SemaphoreType.DMA,
        pltpu.SemaphoreType.DMA,
    ],
)
def _sc_segsum(hp_hbm, src_hbm, dst_hbm, out_hbm, srcb, dstb, rows, accum,
               sem0, sem1, semi0, semi1):
    c = lax.axis_index("c")
    s = lax.axis_index("s")
    wid = c * 16 + s

    _zero_rows(rows.at[0])

    @pl.loop(0, ROWS_PER_TILE // CHUNK)
    def _z(i):
        pltpu.sync_copy(
            rows.at[0], accum.at[pl.ds(s * ROWS_PER_TILE + i * CHUNK, CHUNK)]
        )

    pltpu.sync_copy(dst_hbm.at[pl.ds(wid * NCHUNK, NCHUNK)], dstb)
    row0 = wid * NCHUNK
    pltpu.sync_copy(src_hbm.at[pl.ds(row0, 1)], srcb.at[pl.ds(0, 1)])
    pltpu.sync_copy(src_hbm.at[pl.ds(row0 + 1, 1)], srcb.at[pl.ds(1, 1)])
    plsc.subcore_barrier()

    sems = (sem0, sem1)
    semis = (semi0, semi1)
    pltpu.async_copy(hp_hbm.at[srcb.at[0, 0]], rows.at[0], sem0)

    def _step(k, kb, pb, sn, sr):
        @pl.when(k >= 1)
        def _wait_idx():
            pltpu.make_async_copy(
                src_hbm.at[pl.ds(row0, 1)], srcb.at[pl.ds(sn, 1)], semis[pb]
            ).wait()

        pltpu.async_copy(hp_hbm.at[srcb.at[sn, 0]], rows.at[pb], sems[pb])

        @pl.when(k + 2 <= NCHUNK - 1)
        def _refill():
            pltpu.async_copy(
                src_hbm.at[pl.ds(row0 + k + 2, 1)], srcb.at[pl.ds(sr, 1)],
                semis[kb],
            )

        pltpu.make_async_copy(
            hp_hbm.at[srcb.at[0, 0]], rows.at[kb], sems[kb]
        ).wait()
        pltpu.sync_copy(rows.at[kb], accum.at[dstb.at[k, 0]], add=True)

    @pl.loop(0, NCHUNK - 1)
    def _k(k):
        for r in range(6):
            @pl.when(k % 6 == r)
            def _br(r=r):
                _step(k, r % 2, (r + 1) % 2, (r + 1) % 3, (r + 2) % 3)

    lastb = (NCHUNK - 1) % 2
    pltpu.make_async_copy(
        hp_hbm.at[srcb.at[0, 0]], rows.at[lastb], sems[lastb]
    ).wait()
    pltpu.sync_copy(rows.at[lastb], accum.at[dstb.at[NCHUNK - 1, 0]], add=True)

    plsc.subcore_barrier()
    pltpu.sync_copy(
        accum.at[pl.ds(s * ROWS_PER_TILE, ROWS_PER_TILE)],
        out_hbm.at[pl.ds(c * NPAD + s * ROWS_PER_TILE, ROWS_PER_TILE)],
    )


def _tc1_body(x_ref, w_ref, p0_ref, p1_ref, hp_ref, dinv_ref):
    dinv = lax.rsqrt(p0_ref[...] + p1_ref[...] + 1.0)
    h = jnp.dot(x_ref[...], w_ref[...], preferred_element_type=jnp.float32)
    hp_ref[...] = h * dinv
    dinv_ref[...] = dinv


def _tc2_body(s_ref, hp_ref, dinv_ref, b_ref, w_ref, h2p_ref):
    dinv = dinv_ref[...]
    out1 = dinv * (s_ref[0] + s_ref[1] + hp_ref[...]) + b_ref[...]
    h2 = jnp.dot(out1, w_ref[...], preferred_element_type=jnp.float32)
    h2p_ref[...] = h2 * dinv


def _tc3_body(s_ref, hp_ref, dinv_ref, b_ref, out_ref):
    out_ref[...] = (
        dinv_ref[...] * (s_ref[0] + s_ref[1] + hp_ref[...]) + b_ref[...]
    )


_GRID = (NPAD // R,)
_ROWS = pl.BlockSpec((R, D), lambda i: (i, 0))
_COL = pl.BlockSpec((R, 1), lambda i: (i, 0))
_WMAT = pl.BlockSpec((D, D), lambda i: (0, 0))
_BVEC = pl.BlockSpec((1, D), lambda i: (0, 0))
_PART = pl.BlockSpec((2, R, D), lambda i: (0, i, 0))

_tc1 = pl.pallas_call(
    _tc1_body,
    grid=_GRID,
    in_specs=[_ROWS, _WMAT, _COL, _COL],
    out_specs=[_ROWS, _COL],
    out_shape=[
        jax.ShapeDtypeStruct((NPAD, D), jnp.float32),
        jax.ShapeDtypeStruct((NPAD, 1), jnp.float32),
    ],
)

_tc2 = pl.pallas_call(
    _tc2_body,
    grid=_GRID,
    in_specs=[_PART, _ROWS, _COL, _BVEC, _WMAT],
    out_specs=_ROWS,
    out_shape=jax.ShapeDtypeStruct((NPAD, D), jnp.float32),
)

_tc3 = pl.pallas_call(
    _tc3_body,
    grid=_GRID,
    in_specs=[_PART, _ROWS, _COL, _BVEC],
    out_specs=_ROWS,
    out_shape=jax.ShapeDtypeStruct((NPAD, D), jnp.float32),
)


def kernel(x, edge_index, W1, b1, W2, b2):
    src = edge_index[0].astype(jnp.int32)
    dst = edge_index[1].astype(jnp.int32)
    pad = jnp.full((E_PAD - E,), SENT, jnp.int32)
    src_p = jnp.concatenate([src, pad]).reshape(E_PAD // CHUNK, 1, CHUNK)
    dst_p = jnp.concatenate([dst, pad]).reshape(E_PAD // CHUNK, 1, CHUNK)
    x_pad = jnp.pad(x, ((0, NPAD - N), (0, 0)))
    b1r = b1.reshape(1, D)
    b2r = b2.reshape(1, D)

    degp = _sc_degree(dst_p)
    p0 = degp[:NPAD].reshape(NPAD, 1)
    p1 = degp[NPAD:].reshape(NPAD, 1)

    hp1, dinv = _tc1(x_pad, W1, p0, p1)
    s1 = _sc_segsum(hp1, src_p, dst_p).reshape(2, NPAD, D)
    h2p = _tc2(s1, hp1, dinv, b1r, W2)
    s2 = _sc_segsum(h2p, src_p, dst_p).reshape(2, NPAD, D)
    out2 = _tc3(s2, h2p, dinv, b2r)
    return out2[:N]


# speedup vs baseline: 1.1589x; 1.1589x over previous
import functools

import jax
import jax.numpy as jnp
from jax import lax
from jax.experimental import pallas as pl
from jax.experimental.pallas import tpu as pltpu
from jax.experimental.pallas import tpu_sc as plsc

N = 10000
E = 320000
D = 128

NPAD = 10240
SENT = 10000
NTILES = 32
R = 1024

DCHUNK = 128
EPT = 10240
DNCHUNK = EPT // DCHUNK
E_PAD = EPT * NTILES
DROWS = NPAD // 16

CHUNK = 64
SNCHUNK = E_PAD // CHUNK // 16
A_ROWS = 10112
APT = A_ROWS // 16
HP_HALF = 5120
HP_ROWS = HP_HALF + 8

_MESH = plsc.VectorSubcoreMesh(core_axis_name="c", subcore_axis_name="s")


@functools.partial(
    pl.kernel,
    out_type=jax.ShapeDtypeStruct((2 * NPAD,), jnp.float32),
    mesh=_MESH,
    scratch_types=[
        pltpu.VMEM((DNCHUNK, 1, DCHUNK), jnp.int32),
        pltpu.VMEM((DCHUNK,), jnp.float32),
        pltpu.VMEM((DROWS,), jnp.float32),
        pltpu.VMEM_SHARED((NPAD,), jnp.float32),
    ],
)
def _sc_degree(dst_hbm, out_hbm, dstb, ones, zb, deg):
    c = lax.axis_index("c")
    s = lax.axis_index("s")
    wid = c * 16 + s

    @pl.loop(0, DCHUNK // 16)
    def _o(i):
        ones[pl.ds(i * 16, 16)] = jnp.full((16,), 1.0, jnp.float32)

    @pl.loop(0, DROWS // 16)
    def _z(i):
        zb[pl.ds(i * 16, 16)] = jnp.zeros((16,), jnp.float32)

    pltpu.sync_copy(zb, deg.at[pl.ds(s * DROWS, DROWS)])
    pltpu.sync_copy(dst_hbm.at[pl.ds(wid * DNCHUNK, DNCHUNK)], dstb)
    plsc.subcore_barrier()

    @pl.loop(0, DNCHUNK)
    def _k(k):
        pltpu.sync_copy(ones, deg.at[dstb.at[k, 0]], add=True)

    plsc.subcore_barrier()
    pltpu.sync_copy(
        deg.at[pl.ds(s * DROWS, DROWS)],
        out_hbm.at[pl.ds(c * NPAD + s * DROWS, DROWS)],
    )


@functools.partial(
    pl.kernel,
    out_type=jax.ShapeDtypeStruct((2 * NPAD, D), jnp.float32),
    mesh=_MESH,
    scratch_types=[
        pltpu.VMEM((3, 1, CHUNK), jnp.int32),
        pltpu.VMEM((3, 1, CHUNK), jnp.int32),
        pltpu.VMEM((CHUNK, D), jnp.float32),
        pltpu.VMEM_SHARED((HP_ROWS, D), jnp.float32),
        pltpu.VMEM_SHARED((A_ROWS, D), jnp.float32),
        pltpu.SemaphoreType.DMA,
        pltpu.SemaphoreType.DMA,
        pltpu.SemaphoreType.DMA,
    ],
)
def _sc_segsum(hp_hbm, src_hbm, dst_hbm, out_hbm, srcb, dstb, rows, hps, accum,
               sm0, sm1, sm2):
    c = lax.axis_index("c")
    s = lax.axis_index("s")
    lo = c * HP_HALF
    sems = (sm0, sm1, sm2)

    @pl.loop(0, CHUNK)
    def _zr(r):
        @pl.loop(0, D // 16)
        def _zc(j):
            rows[r, pl.ds(j * 16, 16)] = jnp.zeros((16,), jnp.float32)

    @pl.loop(0, 9)
    def _za(i):
        pltpu.sync_copy(rows, accum.at[pl.ds(s * APT + i * CHUNK, CHUNK)])

    pltpu.sync_copy(rows, accum.at[pl.ds(s * APT + (APT - CHUNK), CHUNK)])

    @pl.when(s == 0)
    def _zh():
        pltpu.sync_copy(rows.at[pl.ds(0, 8)], hps.at[pl.ds(HP_HALF, 8)])

    pltpu.sync_copy(
        hp_hbm.at[pl.ds(lo + s * (HP_HALF // 16), HP_HALF // 16)],
        hps.at[pl.ds(s * (HP_HALF // 16), HP_HALF // 16)],
    )

    base = s * SNCHUNK
    for k0 in range(2):
        pltpu.sync_copy(src_hbm.at[pl.ds(base + k0, 1)], srcb.at[pl.ds(k0, 1)])
        pltpu.sync_copy(dst_hbm.at[pl.ds(base + k0, 1)], dstb.at[pl.ds(k0, 1)])

    plsc.subcore_barrier()

    zrow = HP_HALF + (lax.iota(jnp.int32, 16) & 7)

    def _chunk(k, slot):
        @pl.when(k >= 2)
        def _w():
            pltpu.make_async_copy(
                src_hbm.at[pl.ds(base, 1)], srcb.at[pl.ds(slot, 1)], sems[slot]
            ).wait()
            pltpu.make_async_copy(
                dst_hbm.at[pl.ds(base, 1)], dstb.at[pl.ds(slot, 1)], sems[slot]
            ).wait()

        @pl.loop(0, CHUNK // 16)
        def _m(j):
            v = srcb[slot, 0, pl.ds(j * 16, 16)] - lo
            inh = v.astype(jnp.uint32) < jnp.uint32(HP_HALF)
            srcb[slot, 0, pl.ds(j * 16, 16)] = jnp.where(inh, v, zrow)

        rslot = (slot + 2) % 3

        @pl.when(k + 2 <= SNCHUNK - 1)
        def _r():
            pltpu.async_copy(
                src_hbm.at[pl.ds(base + k + 2, 1)], srcb.at[pl.ds(rslot, 1)],
                sems[rslot],
            )
            pltpu.async_copy(
                dst_hbm.at[pl.ds(base + k + 2, 1)], dstb.at[pl.ds(rslot, 1)],
                sems[rslot],
            )

        pltpu.sync_copy(hps.at[srcb.at[slot, 0]], rows)
        pltpu.sync_copy(rows, accum.at[dstb.at[slot, 0]], add=True)

    @pl.loop(0, SNCHUNK)
    def _k(k):
        for r in range(3):
            @pl.when(k % 3 == r)
            def _br(r=r):
                _chunk(k, r)

    plsc.subcore_barrier()
    pltpu.sync_copy(
        accum.at[pl.ds(s * APT, APT)],
        out_hbm.at[pl.ds(c * NPAD + s * APT, APT)],
    )


def _tc1_body(x_ref, w_ref, p0_ref, p1_ref, hp_ref, dinv_ref):
    dinv = lax.rsqrt(p0_ref[...] + p1_ref[...] + 1.0)
    h = jnp.dot(x_ref[...], w_ref[...], preferred_element_type=jnp.float32)
    hp_ref[...] = h * dinv
    dinv_ref[...] = dinv


def _tc2_body(s_ref, hp_ref, dinv_ref, b_ref, w_ref, h2p_ref):
    dinv = dinv_ref[...]
    out1 = dinv * (s_ref[0] + s_ref[1] + hp_ref[...]) + b_ref[...]
    h2 = jnp.dot(out1, w_ref[...], preferred_element_type=jnp.float32)
    h2p_ref[...] = h2 * dinv


def _tc3_body(s_ref, hp_ref, dinv_ref, b_ref, out_ref):
    out_ref[...] = (
        dinv_ref[...] * (s_ref[0] + s_ref[1] + hp_ref[...]) + b_ref[...]
    )


_GRID = (NPAD // R,)
_ROWS = pl.BlockSpec((R, D), lambda i: (i, 0))
_COL = pl.BlockSpec((R, 1), lambda i: (i, 0))
_WMAT = pl.BlockSpec((D, D), lambda i: (0, 0))
_BVEC = pl.BlockSpec((1, D), lambda i: (0, 0))
_PART = pl.BlockSpec((2, R, D), lambda i: (0, i, 0))

_tc1 = pl.pallas_call(
    _tc1_body,
    grid=_GRID,
    in_specs=[_ROWS, _WMAT, _COL, _COL],
    out_specs=[_ROWS, _COL],
    out_shape=[
        jax.ShapeDtypeStruct((NPAD, D), jnp.float32),
        jax.ShapeDtypeStruct((NPAD, 1), jnp.float32),
    ],
)

_tc2 = pl.pallas_call(
    _tc2_body,
    grid=_GRID,
    in_specs=[_PART, _ROWS, _COL, _BVEC, _WMAT],
    out_specs=_ROWS,
    out_shape=jax.ShapeDtypeStruct((NPAD, D), jnp.float32),
)

_tc3 = pl.pallas_call(
    _tc3_body,
    grid=_GRID,
    in_specs=[_PART, _ROWS, _COL, _BVEC],
    out_specs=_ROWS,
    out_shape=jax.ShapeDtypeStruct((NPAD, D), jnp.float32),
)


def kernel(x, edge_index, W1, b1, W2, b2):
    src = edge_index[0].astype(jnp.int32)
    dst = edge_index[1].astype(jnp.int32)
    pad = jnp.full((E_PAD - E,), SENT, jnp.int32)
    src_p = jnp.concatenate([src, pad])
    dst_p = jnp.concatenate([dst, pad])
    dst_deg = dst_p.reshape(E_PAD // DCHUNK, 1, DCHUNK)
    src_ss = src_p.reshape(E_PAD // CHUNK, 1, CHUNK)
    dst_ss = dst_p.reshape(E_PAD // CHUNK, 1, CHUNK)
    x_pad = jnp.pad(x, ((0, NPAD - N), (0, 0)))
    b1r = b1.reshape(1, D)
    b2r = b2.reshape(1, D)

    degp = _sc_degree(dst_deg)
    p0 = degp[:NPAD].reshape(NPAD, 1)
    p1 = degp[NPAD:].reshape(NPAD, 1)

    hp1, dinv = _tc1(x_pad, W1, p0, p1)
    s1 = _sc_segsum(hp1, src_ss, dst_ss).reshape(2, NPAD, D)
    h2p = _tc2(s1, hp1, dinv, b1r, W2)
    s2 = _sc_segsum(h2p, src_ss, dst_ss).reshape(2, NPAD, D)
    out2 = _tc3(s2, h2p, dinv, b2r)
    return out2[:N]


# speedup vs baseline: 1.3430x; 1.1588x over previous
import functools

import jax
import jax.numpy as jnp
from jax import lax
from jax.experimental import pallas as pl
from jax.experimental.pallas import tpu as pltpu
from jax.experimental.pallas import tpu_sc as plsc

N = 10000
E = 320000
D = 128

NPAD = 10240
SENT = 10000
NTILES = 32
R = 1024

DCHUNK = 128
EPT = 10240
DNCHUNK = EPT // DCHUNK
E_PAD = EPT * NTILES
DROWS = NPAD // 16

CHUNK = 32
SNCHUNK = E_PAD // CHUNK // 16
A_ROWS = 10112
APT = A_ROWS // 16
HP_HALF = 5120
HP_ROWS = HP_HALF + 8

_MESH = plsc.VectorSubcoreMesh(core_axis_name="c", subcore_axis_name="s")


@functools.partial(
    pl.kernel,
    out_type=jax.ShapeDtypeStruct((2 * NPAD,), jnp.float32),
    mesh=_MESH,
    scratch_types=[
        pltpu.VMEM((DNCHUNK, 1, DCHUNK), jnp.int32),
        pltpu.VMEM((DCHUNK,), jnp.float32),
        pltpu.VMEM((DROWS,), jnp.float32),
        pltpu.VMEM_SHARED((NPAD,), jnp.float32),
    ],
)
def _sc_degree(dst_hbm, out_hbm, dstb, ones, zb, deg):
    c = lax.axis_index("c")
    s = lax.axis_index("s")
    wid = c * 16 + s

    @pl.loop(0, DCHUNK // 16)
    def _o(i):
        ones[pl.ds(i * 16, 16)] = jnp.full((16,), 1.0, jnp.float32)

    @pl.loop(0, DROWS // 16)
    def _z(i):
        zb[pl.ds(i * 16, 16)] = jnp.zeros((16,), jnp.float32)

    pltpu.sync_copy(zb, deg.at[pl.ds(s * DROWS, DROWS)])
    pltpu.sync_copy(dst_hbm.at[pl.ds(wid * DNCHUNK, DNCHUNK)], dstb)
    plsc.subcore_barrier()

    @pl.loop(0, DNCHUNK)
    def _k(k):
        pltpu.sync_copy(ones, deg.at[dstb.at[k, 0]], add=True)

    plsc.subcore_barrier()
    pltpu.sync_copy(
        deg.at[pl.ds(s * DROWS, DROWS)],
        out_hbm.at[pl.ds(c * NPAD + s * DROWS, DROWS)],
    )


@functools.partial(
    pl.kernel,
    out_type=jax.ShapeDtypeStruct((2 * NPAD, D), jnp.float32),
    mesh=_MESH,
    scratch_types=[
        pltpu.VMEM((3, 1, CHUNK), jnp.int32),
        pltpu.VMEM((3, 1, CHUNK), jnp.int32),
        pltpu.VMEM((2, CHUNK, D), jnp.float32),
        pltpu.VMEM_SHARED((HP_ROWS, D), jnp.float32),
        pltpu.VMEM_SHARED((A_ROWS, D), jnp.float32),
        pltpu.SemaphoreType.DMA,
        pltpu.SemaphoreType.DMA,
        pltpu.SemaphoreType.DMA,
        pltpu.SemaphoreType.DMA,
        pltpu.SemaphoreType.DMA,
        pltpu.SemaphoreType.DMA,
        pltpu.SemaphoreType.DMA,
    ],
)
def _sc_segsum(hp_hbm, src_hbm, dst_hbm, out_hbm, srcb, dstb, rows, hps, accum,
               sm0, sm1, sm2, gs0, gs1, ss0, ss1):
    c = lax.axis_index("c")
    s = lax.axis_index("s")
    lo = c * HP_HALF
    sems = (sm0, sm1, sm2)
    gsems = (gs0, gs1)
    ssems = (ss0, ss1)

    @pl.loop(0, CHUNK)
    def _zr(r):
        @pl.loop(0, D // 16)
        def _zc(j):
            rows[0, r, pl.ds(j * 16, 16)] = jnp.zeros((16,), jnp.float32)

    @pl.loop(0, APT // CHUNK)
    def _za(i):
        pltpu.sync_copy(rows.at[0], accum.at[pl.ds(s * APT + i * CHUNK, CHUNK)])

    pltpu.sync_copy(rows.at[0], accum.at[pl.ds(s * APT + (APT - CHUNK), CHUNK)])

    @pl.when(s == 0)
    def _zh():
        pltpu.sync_copy(rows.at[0, pl.ds(0, 8)], hps.at[pl.ds(HP_HALF, 8)])

    pltpu.sync_copy(
        hp_hbm.at[pl.ds(lo + s * (HP_HALF // 16), HP_HALF // 16)],
        hps.at[pl.ds(s * (HP_HALF // 16), HP_HALF // 16)],
    )

    base = s * SNCHUNK
    for k0 in range(2):
        pltpu.sync_copy(src_hbm.at[pl.ds(base + k0, 1)], srcb.at[pl.ds(k0, 1)])
        pltpu.sync_copy(dst_hbm.at[pl.ds(base + k0, 1)], dstb.at[pl.ds(k0, 1)])

    plsc.subcore_barrier()

    zrow = HP_HALF + (lax.iota(jnp.int32, 16) & 7)

    def _remap(slot):
        @pl.loop(0, CHUNK // 16)
        def _m(j):
            v = srcb[slot, 0, pl.ds(j * 16, 16)] - lo
            inh = v.astype(jnp.uint32) < jnp.uint32(HP_HALF)
            srcb[slot, 0, pl.ds(j * 16, 16)] = jnp.where(inh, v, zrow)

    _remap(0)
    pltpu.async_copy(hps.at[srcb.at[0, 0]], rows.at[0], gs0)

    def _step(k, b, bn, sn, cs, rslot):
        @pl.when(jnp.logical_and(k >= 1, k + 1 <= SNCHUNK - 1))
        def _wi():
            pltpu.make_async_copy(
                src_hbm.at[pl.ds(base, 1)], srcb.at[pl.ds(sn, 1)], sems[sn]
            ).wait()
            pltpu.make_async_copy(
                dst_hbm.at[pl.ds(base, 1)], dstb.at[pl.ds(sn, 1)], sems[sn]
            ).wait()

        @pl.when(k >= 1)
        def _ws():
            pltpu.make_async_copy(
                rows.at[bn], accum.at[dstb.at[cs, 0]], ssems[bn]
            ).wait()

        @pl.when(k + 1 <= SNCHUNK - 1)
        def _g():
            _remap(sn)
            pltpu.async_copy(hps.at[srcb.at[sn, 0]], rows.at[bn], gsems[bn])

        @pl.when(k + 2 <= SNCHUNK - 1)
        def _r():
            pltpu.async_copy(
                src_hbm.at[pl.ds(base + k + 2, 1)], srcb.at[pl.ds(rslot, 1)],
                sems[rslot],
            )
            pltpu.async_copy(
                dst_hbm.at[pl.ds(base + k + 2, 1)], dstb.at[pl.ds(rslot, 1)],
                sems[rslot],
            )

        pltpu.make_async_copy(
            hps.at[srcb.at[cs, 0]], rows.at[b], gsems[b]
        ).wait()
        pltpu.async_copy(
            rows.at[b], accum.at[dstb.at[cs, 0]], ssems[b], add=True
        )

    @pl.loop(0, SNCHUNK)
    def _k(k):
        for r in range(6):
            @pl.when(k % 6 == r)
            def _br(r=r):
                _step(k, r % 2, (r + 1) % 2, (r + 1) % 3, r % 3, (r + 2) % 3)

    pltpu.make_async_copy(
        rows.at[(SNCHUNK - 1) % 2], accum.at[dstb.at[0, 0]],
        ssems[(SNCHUNK - 1) % 2],
    ).wait()

    plsc.subcore_barrier()
    pltpu.sync_copy(
        accum.at[pl.ds(s * APT, APT)],
        out_hbm.at[pl.ds(c * NPAD + s * APT, APT)],
    )


def _tc1_body(x_ref, w_ref, p0_ref, p1_ref, hp_ref, dinv_ref):
    dinv = lax.rsqrt(p0_ref[...] + p1_ref[...] + 1.0)
    h = jnp.dot(x_ref[...], w_ref[...], preferred_element_type=jnp.float32)
    hp_ref[...] = h * dinv
    dinv_ref[...] = dinv


def _tc2_body(s_ref, hp_ref, dinv_ref, b_ref, w_ref, h2p_ref):
    dinv = dinv_ref[...]
    out1 = dinv * (s_ref[0] + s_ref[1] + hp_ref[...]) + b_ref[...]
    h2 = jnp.dot(out1, w_ref[...], preferred_element_type=jnp.float32)
    h2p_ref[...] = h2 * dinv


def _tc3_body(s_ref, hp_ref, dinv_ref, b_ref, out_ref):
    out_ref[...] = (
        dinv_ref[...] * (s_ref[0] + s_ref[1] + hp_ref[...]) + b_ref[...]
    )


_GRID = (NPAD // R,)
_ROWS = pl.BlockSpec((R, D), lambda i: (i, 0))
_COL = pl.BlockSpec((R, 1), lambda i: (i, 0))
_WMAT = pl.BlockSpec((D, D), lambda i: (0, 0))
_BVEC = pl.BlockSpec((1, D), lambda i: (0, 0))
_PART = pl.BlockSpec((2, R, D), lambda i: (0, i, 0))

_tc1 = pl.pallas_call(
    _tc1_body,
    grid=_GRID,
    in_specs=[_ROWS, _WMAT, _COL, _COL],
    out_specs=[_ROWS, _COL],
    out_shape=[
        jax.ShapeDtypeStruct((NPAD, D), jnp.float32),
        jax.ShapeDtypeStruct((NPAD, 1), jnp.float32),
    ],
)

_tc2 = pl.pallas_call(
    _tc2_body,
    grid=_GRID,
    in_specs=[_PART, _ROWS, _COL, _BVEC, _WMAT],
    out_specs=_ROWS,
    out_shape=jax.ShapeDtypeStruct((NPAD, D), jnp.float32),
)

_tc3 = pl.pallas_call(
    _tc3_body,
    grid=_GRID,
    in_specs=[_PART, _ROWS, _COL, _BVEC],
    out_specs=_ROWS,
    out_shape=jax.ShapeDtypeStruct((NPAD, D), jnp.float32),
)


def kernel(x, edge_index, W1, b1, W2, b2):
    src = edge_index[0].astype(jnp.int32)
    dst = edge_index[1].astype(jnp.int32)
    pad = jnp.full((E_PAD - E,), SENT, jnp.int32)
    src_p = jnp.concatenate([src, pad])
    dst_p = jnp.concatenate([dst, pad])
    dst_deg = dst_p.reshape(E_PAD // DCHUNK, 1, DCHUNK)
    src_ss = src_p.reshape(E_PAD // CHUNK, 1, CHUNK)
    dst_ss = dst_p.reshape(E_PAD // CHUNK, 1, CHUNK)
    x_pad = jnp.pad(x, ((0, NPAD - N), (0, 0)))
    b1r = b1.reshape(1, D)
    b2r = b2.reshape(1, D)

    degp = _sc_degree(dst_deg)
    p0 = degp[:NPAD].reshape(NPAD, 1)
    p1 = degp[NPAD:].reshape(NPAD, 1)

    hp1, dinv = _tc1(x_pad, W1, p0, p1)
    s1 = _sc_segsum(hp1, src_ss, dst_ss).reshape(2, NPAD, D)
    h2p = _tc2(s1, hp1, dinv, b1r, W2)
    s2 = _sc_segsum(h2p, src_ss, dst_ss).reshape(2, NPAD, D)
    out2 = _tc3(s2, h2p, dinv, b2r)
    return out2[:N]


# speedup vs baseline: 1.3484x; 1.0040x over previous
import functools

import jax
import jax.numpy as jnp
from jax import lax
from jax.experimental import pallas as pl
from jax.experimental.pallas import tpu as pltpu
from jax.experimental.pallas import tpu_sc as plsc

N = 10000
E = 320000
D = 128

NPAD = 10240
SENT = 10000
NTILES = 32
R = 1024

DCHUNK = 128
EPT = 10240
DNCHUNK = EPT // DCHUNK
E_PAD = EPT * NTILES
DROWS = NPAD // 16

CHUNK = 32
SNCHUNK = E_PAD // CHUNK // 16
A_ROWS = 10112
APT = A_ROWS // 16
HP_HALF = 5120
HP_ROWS = HP_HALF + 8

_MESH = plsc.VectorSubcoreMesh(core_axis_name="c", subcore_axis_name="s")


@functools.partial(
    pl.kernel,
    out_type=jax.ShapeDtypeStruct((2 * NPAD,), jnp.float32),
    mesh=_MESH,
    scratch_types=[
        pltpu.VMEM((DNCHUNK, 1, DCHUNK), jnp.int32),
        pltpu.VMEM((DCHUNK,), jnp.float32),
        pltpu.VMEM((DROWS,), jnp.float32),
        pltpu.VMEM_SHARED((NPAD,), jnp.float32),
    ],
)
def _sc_degree(dst_hbm, out_hbm, dstb, ones, zb, deg):
    c = lax.axis_index("c")
    s = lax.axis_index("s")
    wid = c * 16 + s

    @pl.loop(0, DCHUNK // 16)
    def _o(i):
        ones[pl.ds(i * 16, 16)] = jnp.full((16,), 1.0, jnp.float32)

    @pl.loop(0, DROWS // 16)
    def _z(i):
        zb[pl.ds(i * 16, 16)] = jnp.zeros((16,), jnp.float32)

    pltpu.sync_copy(zb, deg.at[pl.ds(s * DROWS, DROWS)])
    pltpu.sync_copy(dst_hbm.at[pl.ds(wid * DNCHUNK, DNCHUNK)], dstb)
    plsc.subcore_barrier()

    @pl.loop(0, DNCHUNK)
    def _k(k):
        pltpu.sync_copy(ones, deg.at[dstb.at[k, 0]], add=True)

    plsc.subcore_barrier()
    pltpu.sync_copy(
        deg.at[pl.ds(s * DROWS, DROWS)],
        out_hbm.at[pl.ds(c * NPAD + s * DROWS, DROWS)],
    )


@functools.partial(
    pl.kernel,
    out_type=jax.ShapeDtypeStruct((2 * NPAD, D), jnp.float32),
    mesh=_MESH,
    scratch_types=[
        pltpu.VMEM((3, 1, CHUNK), jnp.int32),
        pltpu.VMEM((3, 1, CHUNK), jnp.int32),
        pltpu.VMEM((2, CHUNK, D), jnp.float32),
        pltpu.VMEM_SHARED((HP_ROWS, D), jnp.float32),
        pltpu.VMEM_SHARED((A_ROWS, D), jnp.float32),
        pltpu.SemaphoreType.DMA,
        pltpu.SemaphoreType.DMA,
        pltpu.SemaphoreType.DMA,
        pltpu.SemaphoreType.DMA,
        pltpu.SemaphoreType.DMA,
    ],
)
def _sc_segsum(hp_hbm, src_hbm, dst_hbm, out_hbm, srcb, dstb, rows, hps, accum,
               sm0, sm1, sm2, gs0, gs1):
    c = lax.axis_index("c")
    s = lax.axis_index("s")
    lo = c * HP_HALF
    sems = (sm0, sm1, sm2)
    gsems = (gs0, gs1)

    @pl.loop(0, CHUNK)
    def _zr(r):
        @pl.loop(0, D // 16)
        def _zc(j):
            rows[0, r, pl.ds(j * 16, 16)] = jnp.zeros((16,), jnp.float32)

    @pl.loop(0, APT // CHUNK)
    def _za(i):
        pltpu.sync_copy(rows.at[0], accum.at[pl.ds(s * APT + i * CHUNK, CHUNK)])

    pltpu.sync_copy(rows.at[0], accum.at[pl.ds(s * APT + (APT - CHUNK), CHUNK)])

    @pl.when(s == 0)
    def _zh():
        pltpu.sync_copy(rows.at[0, pl.ds(0, 8)], hps.at[pl.ds(HP_HALF, 8)])

    pltpu.sync_copy(
        hp_hbm.at[pl.ds(lo + s * (HP_HALF // 16), HP_HALF // 16)],
        hps.at[pl.ds(s * (HP_HALF // 16), HP_HALF // 16)],
    )

    base = s * SNCHUNK
    for k0 in range(2):
        pltpu.sync_copy(src_hbm.at[pl.ds(base + k0, 1)], srcb.at[pl.ds(k0, 1)])
        pltpu.sync_copy(dst_hbm.at[pl.ds(base + k0, 1)], dstb.at[pl.ds(k0, 1)])

    plsc.subcore_barrier()

    zrow = HP_HALF + (lax.iota(jnp.int32, 16) & 7)

    def _remap(slot):
        @pl.loop(0, CHUNK // 16)
        def _m(j):
            v = srcb[slot, 0, pl.ds(j * 16, 16)] - lo
            inh = v.astype(jnp.uint32) < jnp.uint32(HP_HALF)
            srcb[slot, 0, pl.ds(j * 16, 16)] = jnp.where(inh, v, zrow)

    _remap(0)
    pltpu.async_copy(hps.at[srcb.at[0, 0]], rows.at[0], gs0)

    def _step(k, b, bn, sn, cs, rslot):
        @pl.when(jnp.logical_and(k >= 1, k + 1 <= SNCHUNK - 1))
        def _wi():
            pltpu.make_async_copy(
                src_hbm.at[pl.ds(base, 1)], srcb.at[pl.ds(sn, 1)], sems[sn]
            ).wait()
            pltpu.make_async_copy(
                dst_hbm.at[pl.ds(base, 1)], dstb.at[pl.ds(sn, 1)], sems[sn]
            ).wait()

        @pl.when(k + 1 <= SNCHUNK - 1)
        def _g():
            _remap(sn)
            pltpu.async_copy(hps.at[srcb.at[sn, 0]], rows.at[bn], gsems[bn])

        @pl.when(k + 2 <= SNCHUNK - 1)
        def _r():
            pltpu.async_copy(
                src_hbm.at[pl.ds(base + k + 2, 1)], srcb.at[pl.ds(rslot, 1)],
                sems[rslot],
            )
            pltpu.async_copy(
                dst_hbm.at[pl.ds(base + k + 2, 1)], dstb.at[pl.ds(rslot, 1)],
                sems[rslot],
            )

        pltpu.make_async_copy(
            hps.at[srcb.at[cs, 0]], rows.at[b], gsems[b]
        ).wait()
        pltpu.sync_copy(rows.at[b], accum.at[dstb.at[cs, 0]], add=True)

    @pl.loop(0, SNCHUNK)
    def _k(k):
        for r in range(6):
            @pl.when(k % 6 == r)
            def _br(r=r):
                _step(k, r % 2, (r + 1) % 2, (r + 1) % 3, r % 3, (r + 2) % 3)

    plsc.subcore_barrier()
    pltpu.sync_copy(
        accum.at[pl.ds(s * APT, APT)],
        out_hbm.at[pl.ds(c * NPAD + s * APT, APT)],
    )


def _tc1_body(x_ref, w_ref, p0_ref, p1_ref, hp_ref, dinv_ref):
    dinv = lax.rsqrt(p0_ref[...] + p1_ref[...] + 1.0)
    h = jnp.dot(x_ref[...], w_ref[...], preferred_element_type=jnp.float32)
    hp_ref[...] = h * dinv
    dinv_ref[...] = dinv


def _tc2_body(s_ref, hp_ref, dinv_ref, b_ref, w_ref, h2p_ref):
    dinv = dinv_ref[...]
    out1 = dinv * (s_ref[0] + s_ref[1] + hp_ref[...]) + b_ref[...]
    h2 = jnp.dot(out1, w_ref[...], preferred_element_type=jnp.float32)
    h2p_ref[...] = h2 * dinv


def _tc3_body(s_ref, hp_ref, dinv_ref, b_ref, out_ref):
    out_ref[...] = (
        dinv_ref[...] * (s_ref[0] + s_ref[1] + hp_ref[...]) + b_ref[...]
    )


_GRID = (NPAD // R,)
_ROWS = pl.BlockSpec((R, D), lambda i: (i, 0))
_COL = pl.BlockSpec((R, 1), lambda i: (i, 0))
_WMAT = pl.BlockSpec((D, D), lambda i: (0, 0))
_BVEC = pl.BlockSpec((1, D), lambda i: (0, 0))
_PART = pl.BlockSpec((2, R, D), lambda i: (0, i, 0))

_tc1 = pl.pallas_call(
    _tc1_body,
    grid=_GRID,
    in_specs=[_ROWS, _WMAT, _COL, _COL],
    out_specs=[_ROWS, _COL],
    out_shape=[
        jax.ShapeDtypeStruct((NPAD, D), jnp.float32),
        jax.ShapeDtypeStruct((NPAD, 1), jnp.float32),
    ],
)

_tc2 = pl.pallas_call(
    _tc2_body,
    grid=_GRID,
    in_specs=[_PART, _ROWS, _COL, _BVEC, _WMAT],
    out_specs=_ROWS,
    out_shape=jax.ShapeDtypeStruct((NPAD, D), jnp.float32),
)

_tc3 = pl.pallas_call(
    _tc3_body,
    grid=_GRID,
    in_specs=[_PART, _ROWS, _COL, _BVEC],
    out_specs=_ROWS,
    out_shape=jax.ShapeDtypeStruct((NPAD, D), jnp.float32),
)


def kernel(x, edge_index, W1, b1, W2, b2):
    src = edge_index[0].astype(jnp.int32)
    dst = edge_index[1].astype(jnp.int32)
    pad = jnp.full((E_PAD - E,), SENT, jnp.int32)
    src_p = jnp.concatenate([src, pad])
    dst_p = jnp.concatenate([dst, pad])
    dst_deg = dst_p.reshape(E_PAD // DCHUNK, 1, DCHUNK)
    src_ss = src_p.reshape(E_PAD // CHUNK, 1, CHUNK)
    dst_ss = dst_p.reshape(E_PAD // CHUNK, 1, CHUNK)
    x_pad = jnp.pad(x, ((0, NPAD - N), (0, 0)))
    b1r = b1.reshape(1, D)
    b2r = b2.reshape(1, D)

    degp = _sc_degree(dst_deg)
    p0 = degp[:NPAD].reshape(NPAD, 1)
    p1 = degp[NPAD:].reshape(NPAD, 1)

    hp1, dinv = _tc1(x_pad, W1, p0, p1)
    s1 = _sc_segsum(hp1, src_ss, dst_ss).reshape(2, NPAD, D)
    h2p = _tc2(s1, hp1, dinv, b1r, W2)
    s2 = _sc_segsum(h2p, src_ss, dst_ss).reshape(2, NPAD, D)
    out2 = _tc3(s2, h2p, dinv, b2r)
    return out2[:N]
